# baseline (speedup 1.0000x reference)
import functools

import jax
import jax.numpy as jnp
from jax import lax
from jax.experimental import pallas as pl
from jax.experimental.pallas import tpu as pltpu
from jax.experimental.pallas import tpu_sc as plsc

N = 10000
NP = 10240
E = 160000
EP = 163840
DH = 128
NC, NS, L = 2, 16, 16
CHUNK = 128
NSEG = 2
NBUF = 2
ROWS_PER_TILE = NP // NS
BR = 256


def _mesh():
    return plsc.VectorSubcoreMesh(core_axis_name="c", subcore_axis_name="s")


def _sc_degree(dst2d):
    half_ch = (EP // NC) // CHUNK
    nch = half_ch // NS
    stripe_ch = ROWS_PER_TILE // CHUNK

    def body(dst_hbm, out0, out1, dst_all, ones_v, buf_v, acc_sh, sem):
        c = lax.axis_index("c")
        t = lax.axis_index("s")

        pltpu.sync_copy(dst_hbm.at[pl.ds((c * NS + t) * nch, nch)], dst_all)

        one = jnp.full((L,), 1.0, jnp.float32)
        zero = jnp.zeros((L,), jnp.float32)

        def init_ones(i, _):
            for j in range(DH // L):
                ones_v[i, pl.ds(j * L, L)] = one
                buf_v[i, pl.ds(j * L, L)] = zero
            return 0
        lax.fori_loop(0, CHUNK, init_ones, 0)

        for j in range(stripe_ch):
            pltpu.sync_copy(
                buf_v, acc_sh.at[pl.ds(t * ROWS_PER_TILE + j * CHUNK, CHUNK)])
        plsc.subcore_barrier()

        def fire(k, _):
            pltpu.async_copy(ones_v, acc_sh.at[dst_all.at[k]], sem, add=True)
            return 0
        lax.fori_loop(0, nch, fire, 0)

        def drain(k, _):
            pltpu.make_async_copy(ones_v, acc_sh.at[dst_all.at[k]], sem).wait()
            return 0
        lax.fori_loop(0, nch, drain, 0)
        plsc.subcore_barrier()

        def make_out(out_hbm):
            def copy_out(j, _):
                base = t * ROWS_PER_TILE + j * CHUNK
                pltpu.sync_copy(acc_sh.at[pl.ds(base, CHUNK)], buf_v)
                pltpu.sync_copy(buf_v, out_hbm.at[pl.ds(base, CHUNK)])
                return 0
            return copy_out

        @pl.when(c == 0)
        def _():
            lax.fori_loop(0, stripe_ch, make_out(out0), 0)

        @pl.when(c == 1)
        def _():
            lax.fori_loop(0, stripe_ch, make_out(out1), 0)

    f = pl.kernel(
        body,
        out_type=(jax.ShapeDtypeStruct((NP, DH), jnp.float32),
                  jax.ShapeDtypeStruct((NP, DH), jnp.float32)),
        mesh=_mesh(),
        scratch_types=[
            pltpu.VMEM((EP // NC // NS // CHUNK, CHUNK), jnp.int32),
            pltpu.VMEM((CHUNK, DH), jnp.float32),
            pltpu.VMEM((CHUNK, DH), jnp.float32),
            pltpu.VMEM_SHARED((NP, DH), jnp.float32),
            pltpu.SemaphoreType.DMA,
        ],
    )
    return f(dst2d)


def _sc_scatter(h0, h1, src2d, dst2d):
    nch = EP // NS // CHUNK
    stripe_ch = ROWS_PER_TILE // CHUNK

    def body(h0_hbm, h1_hbm, src_hbm, dst_hbm, out0, out1,
             src_all, dst_all, bufs, acc_sh, gsems, ssems):
        c = lax.axis_index("c")
        t = lax.axis_index("s")

        zero = jnp.zeros((L,), jnp.float32)

        def init_rows(i, _):
            for j in range(DH // L):
                bufs[0][i, pl.ds(j * L, L)] = zero
            return 0
        lax.fori_loop(0, CHUNK, init_rows, 0)
        for j in range(stripe_ch):
            pltpu.sync_copy(
                bufs[0],
                acc_sh.at[pl.ds(t * ROWS_PER_TILE + j * CHUNK, CHUNK)])
        plsc.subcore_barrier()

        seg = nch // NSEG

        def make_loop(h_hbm):
            def _loop():
                for s in range(NSEG):
                    pltpu.sync_copy(
                        src_hbm.at[pl.ds(t * nch + s * seg, seg)], src_all)
                    pltpu.sync_copy(
                        dst_hbm.at[pl.ds(t * nch + s * seg, seg)], dst_all)
                    for b in range(NBUF):
                        pltpu.async_copy(
                            h_hbm.at[src_all.at[b]], bufs[b], gsems[b])

                    def step(g, _):
                        k0 = g * NBUF
                        for b in range(NBUF):
                            pltpu.make_async_copy(
                                h_hbm.at[src_all.at[k0 + b]],
                                bufs[b], gsems[b]).wait()
                            pltpu.async_copy(
                                bufs[b], acc_sh.at[dst_all.at[k0 + b]],
                                ssems[b], add=True)
                        for b in range(NBUF):
                            @pl.when(k0 + b + NBUF < seg)
                            def _():
                                pltpu.make_async_copy(
                                    bufs[b], acc_sh.at[dst_all.at[k0 + b]],
                                    ssems[b]).wait()
                                pltpu.async_copy(
                                    h_hbm.at[src_all.at[k0 + b + NBUF]],
                                    bufs[b], gsems[b])
                        return 0
                    lax.fori_loop(0, seg // NBUF, step, 0)
                    for b in range(NBUF):
                        pltpu.make_async_copy(
                            bufs[b], acc_sh.at[dst_all.at[seg - NBUF + b]],
                            ssems[b]).wait()
            return _loop

        @pl.when(c == 0)
        def _():
            make_loop(h0_hbm)()

        @pl.when(c == 1)
        def _():
            make_loop(h1_hbm)()

        plsc.subcore_barrier()

        def make_out(out_hbm):
            def copy_out(j, _):
                base = t * ROWS_PER_TILE + j * CHUNK
                pltpu.sync_copy(acc_sh.at[pl.ds(base, CHUNK)], bufs[0])
                pltpu.sync_copy(bufs[0], out_hbm.at[pl.ds(base, CHUNK)])
                return 0
            return copy_out

        @pl.when(c == 0)
        def _():
            lax.fori_loop(0, stripe_ch, make_out(out0), 0)

        @pl.when(c == 1)
        def _():
            lax.fori_loop(0, stripe_ch, make_out(out1), 0)

    f = pl.kernel(
        body,
        out_type=(jax.ShapeDtypeStruct((NP, DH), jnp.float32),
                  jax.ShapeDtypeStruct((NP, DH), jnp.float32)),
        mesh=_mesh(),
        scratch_types=[
            pltpu.VMEM((EP // NS // CHUNK // NSEG, CHUNK), jnp.int32),
            pltpu.VMEM((EP // NS // CHUNK // NSEG, CHUNK), jnp.int32),
            [pltpu.VMEM((CHUNK, DH), jnp.float32) for _ in range(NBUF)],
            pltpu.VMEM_SHARED((NP, DH), jnp.float32),
            [pltpu.SemaphoreType.DMA for _ in range(NBUF)],
            [pltpu.SemaphoreType.DMA for _ in range(NBUF)],
        ],
    )
    return f(h0, h1, src2d, dst2d)


def _dis(dp0_blk, dp1_blk):
    deg = 1.0 + dp0_blk[:, 0:1] + dp1_blk[:, 0:1]
    return lax.rsqrt(deg)


def _tc_layer1_body(x_ref, w_ref, dp0_ref, dp1_ref, o0_ref, o1_ref):
    s = _dis(dp0_ref[...], dp1_ref[...])
    xh = jnp.dot(x_ref[...], w_ref[...], preferred_element_type=jnp.float32)
    xs = xh * s
    o0_ref[...] = xs[:, :DH]
    o1_ref[...] = xs[:, DH:]


def _tc_layer1(xp, W1, dp0, dp1):
    grid = (NP // BR,)
    return pl.pallas_call(
        _tc_layer1_body,
        grid=grid,
        in_specs=[
            pl.BlockSpec((BR, 256), lambda i: (i, 0)),
            pl.BlockSpec((256, 256), lambda i: (0, 0)),
            pl.BlockSpec((BR, DH), lambda i: (i, 0)),
            pl.BlockSpec((BR, DH), lambda i: (i, 0)),
        ],
        out_specs=(pl.BlockSpec((BR, DH), lambda i: (i, 0)),
                   pl.BlockSpec((BR, DH), lambda i: (i, 0))),
        out_shape=(jax.ShapeDtypeStruct((NP, DH), jnp.float32),
                   jax.ShapeDtypeStruct((NP, DH), jnp.float32)),
    )(xp, W1, dp0, dp1)


def _tc_mid_body(s0_ref, s1_ref, x0_ref, x1_ref, dp0_ref, dp1_ref, b_ref,
                 o0_ref, o1_ref):
    s = _dis(dp0_ref[...], dp1_ref[...])
    pre0 = s * (s0_ref[...] + x0_ref[...]) + b_ref[0:1, :]
    pre1 = s * (s1_ref[...] + x1_ref[...]) + b_ref[1:2, :]
    o0_ref[...] = s * jnp.maximum(pre0, 0.0)
    o1_ref[...] = s * jnp.maximum(pre1, 0.0)


def _tc_mid(s0, s1, x0, x1, dp0, dp1, b1):
    grid = (NP // BR,)
    b2 = b1.reshape(2, DH)
    blk = lambda i: (i, 0)
    return pl.pallas_call(
        _tc_mid_body,
        grid=grid,
        in_specs=[
            pl.BlockSpec((BR, DH), blk),
            pl.BlockSpec((BR, DH), blk),
            pl.BlockSpec((BR, DH), blk),
            pl.BlockSpec((BR, DH), blk),
            pl.BlockSpec((BR, DH), blk),
            pl.BlockSpec((BR, DH), blk),
            pl.BlockSpec((2, DH), lambda i: (0, 0)),
        ],
        out_specs=(pl.BlockSpec((BR, DH), blk),
                   pl.BlockSpec((BR, DH), blk)),
        out_shape=(jax.ShapeDtypeStruct((NP, DH), jnp.float32),
                   jax.ShapeDtypeStruct((NP, DH), jnp.float32)),
    )(s0, s1, x0, x1, dp0, dp1, b2)


def _tc_final_body(t0_ref, t1_ref, h0_ref, h1_ref, dp0_ref, dp1_ref,
                   wmu_ref, bmu_ref, wlv_ref, blv_ref, mu_ref, lv_ref):
    s = _dis(dp0_ref[...], dp1_ref[...])
    c0 = s * (t0_ref[...] + h0_ref[...])
    c1 = s * (t1_ref[...] + h1_ref[...])
    wmu = wmu_ref[...]
    wlv = wlv_ref[...]
    mu_ref[...] = (jnp.dot(c0, wmu[:DH], preferred_element_type=jnp.float32)
                   + jnp.dot(c1, wmu[DH:], preferred_element_type=jnp.float32)
                   + bmu_ref[0:1, :])
    lv_ref[...] = (jnp.dot(c0, wlv[:DH], preferred_element_type=jnp.float32)
                   + jnp.dot(c1, wlv[DH:], preferred_element_type=jnp.float32)
                   + blv_ref[0:1, :])


def _tc_final(t0, t1, h0, h1, dp0, dp1, W_mu, b_mu, W_logvar, b_logvar):
    grid = (NP // BR,)
    blk = lambda i: (i, 0)
    return pl.pallas_call(
        _tc_final_body,
        grid=grid,
        in_specs=[
            pl.BlockSpec((BR, DH), blk),
            pl.BlockSpec((BR, DH), blk),
            pl.BlockSpec((BR, DH), blk),
            pl.BlockSpec((BR, DH), blk),
            pl.BlockSpec((BR, DH), blk),
            pl.BlockSpec((BR, DH), blk),
            pl.BlockSpec((256, DH), lambda i: (0, 0)),
            pl.BlockSpec((1, DH), lambda i: (0, 0)),
            pl.BlockSpec((256, DH), lambda i: (0, 0)),
            pl.BlockSpec((1, DH), lambda i: (0, 0)),
        ],
        out_specs=(pl.BlockSpec((BR, DH), blk),
                   pl.BlockSpec((BR, DH), blk)),
        out_shape=(jax.ShapeDtypeStruct((NP, DH), jnp.float32),
                   jax.ShapeDtypeStruct((NP, DH), jnp.float32)),
    )(t0, t1, h0, h1, dp0, dp1, W_mu, b_mu.reshape(1, DH),
      W_logvar, b_logvar.reshape(1, DH))


def kernel(x, edge_index, W1, b1, W_mu, b_mu, W_logvar, b_logvar):
    src = edge_index[0].astype(jnp.int32)
    dst = edge_index[1].astype(jnp.int32)
    pad = EP - E
    src2d = jnp.concatenate([src, jnp.zeros((pad,), jnp.int32)]).reshape(
        EP // CHUNK, CHUNK)
    dst2d = jnp.concatenate([dst, jnp.full((pad,), N, jnp.int32)]).reshape(
        EP // CHUNK, CHUNK)
    xp = jnp.concatenate([x, jnp.zeros((NP - N, x.shape[1]), x.dtype)])

    dp0, dp1 = _sc_degree(dst2d)
    x0, x1 = _tc_layer1(xp, W1, dp0, dp1)
    s0, s1 = _sc_scatter(x0, x1, src2d, dst2d)
    h0, h1 = _tc_mid(s0, s1, x0, x1, dp0, dp1, b1)
    t0, t1 = _sc_scatter(h0, h1, src2d, dst2d)
    mu, lv = _tc_final(t0, t1, h0, h1, dp0, dp1, W_mu, b_mu, W_logvar, b_logvar)
    return mu[:N], lv[:N]


# speedup vs baseline: 1.0664x; 1.0664x over previous
import functools

import jax
import jax.numpy as jnp
from jax import lax
from jax.experimental import pallas as pl
from jax.experimental.pallas import tpu as pltpu
from jax.experimental.pallas import tpu_sc as plsc

N = 10000
NP = 10240
E = 160000
EP = 163840
DH = 128
NC, NS, L = 2, 16, 16
CHUNK = 128
NSEG = 2
NBUF = 2
ROWS_PER_TILE = NP // NS
BR = 256


def _mesh():
    return plsc.VectorSubcoreMesh(core_axis_name="c", subcore_axis_name="s")


def _sc_degree(dst2d):
    half_ch = (EP // NC) // CHUNK
    nch = half_ch // NS
    stripe_ch = ROWS_PER_TILE // CHUNK

    def body(dst_hbm, out0, out1, dst_all, ones_v, buf_v, acc_sh, sem):
        c = lax.axis_index("c")
        t = lax.axis_index("s")

        pltpu.sync_copy(dst_hbm.at[pl.ds((c * NS + t) * nch, nch)], dst_all)

        one = jnp.full((L,), 1.0, jnp.float32)
        zero = jnp.zeros((L,), jnp.float32)

        def init_ones(i, _):
            for j in range(DH // L):
                ones_v[i, pl.ds(j * L, L)] = one
                buf_v[i, pl.ds(j * L, L)] = zero
            return 0
        lax.fori_loop(0, CHUNK, init_ones, 0)

        for j in range(stripe_ch):
            pltpu.sync_copy(
                buf_v, acc_sh.at[pl.ds(t * ROWS_PER_TILE + j * CHUNK, CHUNK)])
        plsc.subcore_barrier()

        def fire(k, _):
            pltpu.async_copy(ones_v, acc_sh.at[dst_all.at[k]], sem, add=True)
            return 0
        lax.fori_loop(0, nch, fire, 0)

        def drain(k, _):
            pltpu.make_async_copy(ones_v, acc_sh.at[dst_all.at[k]], sem).wait()
            return 0
        lax.fori_loop(0, nch, drain, 0)
        plsc.subcore_barrier()

        def make_out(out_hbm):
            def copy_out(j, _):
                base = t * ROWS_PER_TILE + j * CHUNK
                pltpu.sync_copy(acc_sh.at[pl.ds(base, CHUNK)], buf_v)
                pltpu.sync_copy(buf_v, out_hbm.at[pl.ds(base, CHUNK)])
                return 0
            return copy_out

        @pl.when(c == 0)
        def _():
            lax.fori_loop(0, stripe_ch, make_out(out0), 0)

        @pl.when(c == 1)
        def _():
            lax.fori_loop(0, stripe_ch, make_out(out1), 0)

    f = pl.kernel(
        body,
        out_type=(jax.ShapeDtypeStruct((NP, DH), jnp.float32),
                  jax.ShapeDtypeStruct((NP, DH), jnp.float32)),
        mesh=_mesh(),
        scratch_types=[
            pltpu.VMEM((EP // NC // NS // CHUNK, CHUNK), jnp.int32),
            pltpu.VMEM((CHUNK, DH), jnp.float32),
            pltpu.VMEM((CHUNK, DH), jnp.float32),
            pltpu.VMEM_SHARED((NP, DH), jnp.float32),
            pltpu.SemaphoreType.DMA,
        ],
    )
    return f(dst2d)


def _sc_scatter(h0, h1, src2d, dst2d):
    nch = EP // NS // CHUNK
    stripe_ch = ROWS_PER_TILE // CHUNK

    def body(h0_hbm, h1_hbm, src_hbm, dst_hbm, out0, out1,
             src_all, dst_all, bufs, acc_sh, gsems, ssems):
        c = lax.axis_index("c")
        t = lax.axis_index("s")

        zero = jnp.zeros((L,), jnp.float32)

        def init_rows(i, _):
            for j in range(DH // L):
                bufs[0][i, pl.ds(j * L, L)] = zero
            return 0
        lax.fori_loop(0, CHUNK, init_rows, 0)
        for j in range(stripe_ch):
            pltpu.sync_copy(
                bufs[0],
                acc_sh.at[pl.ds(t * ROWS_PER_TILE + j * CHUNK, CHUNK)])
        plsc.subcore_barrier()

        seg = nch // NSEG

        def make_loop(h_hbm):
            def _loop():
                for s in range(NSEG):
                    pltpu.sync_copy(
                        src_hbm.at[pl.ds(t * nch + s * seg, seg)], src_all)
                    pltpu.sync_copy(
                        dst_hbm.at[pl.ds(t * nch + s * seg, seg)], dst_all)
                    for b in range(NBUF):
                        pltpu.async_copy(
                            h_hbm.at[src_all.at[b]], bufs[b], gsems[b])

                    def step(g, _):
                        k0 = g * NBUF
                        for b in range(NBUF):
                            pltpu.make_async_copy(
                                h_hbm.at[src_all.at[k0 + b]],
                                bufs[b], gsems[b]).wait()
                            pltpu.sync_copy(
                                bufs[b], acc_sh.at[dst_all.at[k0 + b]],
                                add=True)

                            @pl.when(k0 + b + NBUF < seg)
                            def _():
                                pltpu.async_copy(
                                    h_hbm.at[src_all.at[k0 + b + NBUF]],
                                    bufs[b], gsems[b])
                        return 0
                    lax.fori_loop(0, seg // NBUF, step, 0)
            return _loop

        @pl.when(c == 0)
        def _():
            make_loop(h0_hbm)()

        @pl.when(c == 1)
        def _():
            make_loop(h1_hbm)()

        plsc.subcore_barrier()

        def make_out(out_hbm):
            def copy_out(j, _):
                base = t * ROWS_PER_TILE + j * CHUNK
                pltpu.sync_copy(acc_sh.at[pl.ds(base, CHUNK)], bufs[0])
                pltpu.sync_copy(bufs[0], out_hbm.at[pl.ds(base, CHUNK)])
                return 0
            return copy_out

        @pl.when(c == 0)
        def _():
            lax.fori_loop(0, stripe_ch, make_out(out0), 0)

        @pl.when(c == 1)
        def _():
            lax.fori_loop(0, stripe_ch, make_out(out1), 0)

    f = pl.kernel(
        body,
        out_type=(jax.ShapeDtypeStruct((NP, DH), jnp.float32),
                  jax.ShapeDtypeStruct((NP, DH), jnp.float32)),
        mesh=_mesh(),
        scratch_types=[
            pltpu.VMEM((EP // NS // CHUNK // NSEG, CHUNK), jnp.int32),
            pltpu.VMEM((EP // NS // CHUNK // NSEG, CHUNK), jnp.int32),
            [pltpu.VMEM((CHUNK, DH), jnp.float32) for _ in range(NBUF)],
            pltpu.VMEM_SHARED((NP, DH), jnp.float32),
            [pltpu.SemaphoreType.DMA for _ in range(NBUF)],
            [pltpu.SemaphoreType.DMA for _ in range(NBUF)],
        ],
    )
    return f(h0, h1, src2d, dst2d)


def _dis(dp0_blk, dp1_blk):
    deg = 1.0 + dp0_blk[:, 0:1] + dp1_blk[:, 0:1]
    return lax.rsqrt(deg)


def _tc_layer1_body(x_ref, w_ref, dp0_ref, dp1_ref, o0_ref, o1_ref):
    s = _dis(dp0_ref[...], dp1_ref[...])
    xh = jnp.dot(x_ref[...], w_ref[...], preferred_element_type=jnp.float32)
    xs = xh * s
    o0_ref[...] = xs[:, :DH]
    o1_ref[...] = xs[:, DH:]


def _tc_layer1(xp, W1, dp0, dp1):
    grid = (NP // BR,)
    return pl.pallas_call(
        _tc_layer1_body,
        grid=grid,
        in_specs=[
            pl.BlockSpec((BR, 256), lambda i: (i, 0)),
            pl.BlockSpec((256, 256), lambda i: (0, 0)),
            pl.BlockSpec((BR, DH), lambda i: (i, 0)),
            pl.BlockSpec((BR, DH), lambda i: (i, 0)),
        ],
        out_specs=(pl.BlockSpec((BR, DH), lambda i: (i, 0)),
                   pl.BlockSpec((BR, DH), lambda i: (i, 0))),
        out_shape=(jax.ShapeDtypeStruct((NP, DH), jnp.float32),
                   jax.ShapeDtypeStruct((NP, DH), jnp.float32)),
    )(xp, W1, dp0, dp1)


def _tc_mid_body(s0_ref, s1_ref, x0_ref, x1_ref, dp0_ref, dp1_ref, b_ref,
                 o0_ref, o1_ref):
    s = _dis(dp0_ref[...], dp1_ref[...])
    pre0 = s * (s0_ref[...] + x0_ref[...]) + b_ref[0:1, :]
    pre1 = s * (s1_ref[...] + x1_ref[...]) + b_ref[1:2, :]
    o0_ref[...] = s * jnp.maximum(pre0, 0.0)
    o1_ref[...] = s * jnp.maximum(pre1, 0.0)


def _tc_mid(s0, s1, x0, x1, dp0, dp1, b1):
    grid = (NP // BR,)
    b2 = b1.reshape(2, DH)
    blk = lambda i: (i, 0)
    return pl.pallas_call(
        _tc_mid_body,
        grid=grid,
        in_specs=[
            pl.BlockSpec((BR, DH), blk),
            pl.BlockSpec((BR, DH), blk),
            pl.BlockSpec((BR, DH), blk),
            pl.BlockSpec((BR, DH), blk),
            pl.BlockSpec((BR, DH), blk),
            pl.BlockSpec((BR, DH), blk),
            pl.BlockSpec((2, DH), lambda i: (0, 0)),
        ],
        out_specs=(pl.BlockSpec((BR, DH), blk),
                   pl.BlockSpec((BR, DH), blk)),
        out_shape=(jax.ShapeDtypeStruct((NP, DH), jnp.float32),
                   jax.ShapeDtypeStruct((NP, DH), jnp.float32)),
    )(s0, s1, x0, x1, dp0, dp1, b2)


def _tc_final_body(t0_ref, t1_ref, h0_ref, h1_ref, dp0_ref, dp1_ref,
                   wmu_ref, bmu_ref, wlv_ref, blv_ref, mu_ref, lv_ref):
    s = _dis(dp0_ref[...], dp1_ref[...])
    c0 = s * (t0_ref[...] + h0_ref[...])
    c1 = s * (t1_ref[...] + h1_ref[...])
    wmu = wmu_ref[...]
    wlv = wlv_ref[...]
    mu_ref[...] = (jnp.dot(c0, wmu[:DH], preferred_element_type=jnp.float32)
                   + jnp.dot(c1, wmu[DH:], preferred_element_type=jnp.float32)
                   + bmu_ref[0:1, :])
    lv_ref[...] = (jnp.dot(c0, wlv[:DH], preferred_element_type=jnp.float32)
                   + jnp.dot(c1, wlv[DH:], preferred_element_type=jnp.float32)
                   + blv_ref[0:1, :])


def _tc_final(t0, t1, h0, h1, dp0, dp1, W_mu, b_mu, W_logvar, b_logvar):
    grid = (NP // BR,)
    blk = lambda i: (i, 0)
    return pl.pallas_call(
        _tc_final_body,
        grid=grid,
        in_specs=[
            pl.BlockSpec((BR, DH), blk),
            pl.BlockSpec((BR, DH), blk),
            pl.BlockSpec((BR, DH), blk),
            pl.BlockSpec((BR, DH), blk),
            pl.BlockSpec((BR, DH), blk),
            pl.BlockSpec((BR, DH), blk),
            pl.BlockSpec((256, DH), lambda i: (0, 0)),
            pl.BlockSpec((1, DH), lambda i: (0, 0)),
            pl.BlockSpec((256, DH), lambda i: (0, 0)),
            pl.BlockSpec((1, DH), lambda i: (0, 0)),
        ],
        out_specs=(pl.BlockSpec((BR, DH), blk),
                   pl.BlockSpec((BR, DH), blk)),
        out_shape=(jax.ShapeDtypeStruct((NP, DH), jnp.float32),
                   jax.ShapeDtypeStruct((NP, DH), jnp.float32)),
    )(t0, t1, h0, h1, dp0, dp1, W_mu, b_mu.reshape(1, DH),
      W_logvar, b_logvar.reshape(1, DH))


def kernel(x, edge_index, W1, b1, W_mu, b_mu, W_logvar, b_logvar):
    src = edge_index[0].astype(jnp.int32)
    dst = edge_index[1].astype(jnp.int32)
    pad = EP - E
    src2d = jnp.concatenate([src, jnp.zeros((pad,), jnp.int32)]).reshape(
        EP // CHUNK, CHUNK)
    dst2d = jnp.concatenate([dst, jnp.full((pad,), N, jnp.int32)]).reshape(
        EP // CHUNK, CHUNK)
    xp = jnp.concatenate([x, jnp.zeros((NP - N, x.shape[1]), x.dtype)])

    dp0, dp1 = _sc_degree(dst2d)
    x0, x1 = _tc_layer1(xp, W1, dp0, dp1)
    s0, s1 = _sc_scatter(x0, x1, src2d, dst2d)
    h0, h1 = _tc_mid(s0, s1, x0, x1, dp0, dp1, b1)
    t0, t1 = _sc_scatter(h0, h1, src2d, dst2d)
    mu, lv = _tc_final(t0, t1, h0, h1, dp0, dp1, W_mu, b_mu, W_logvar, b_logvar)
    return mu[:N], lv[:N]


# speedup vs baseline: 1.0713x; 1.0045x over previous
import functools

import jax
import jax.numpy as jnp
from jax import lax
from jax.experimental import pallas as pl
from jax.experimental.pallas import tpu as pltpu
from jax.experimental.pallas import tpu_sc as plsc

N = 10000
NP = 10240
E = 160000
EP = 163840
DH = 128
NC, NS, L = 2, 16, 16
CHUNK = 128
NSEG = 2
NBUF = 2
ROWS_PER_TILE = NP // NS
BR = 256


def _mesh():
    return plsc.VectorSubcoreMesh(core_axis_name="c", subcore_axis_name="s")


def _sc_degree(dst2d):
    half_ch = (EP // NC) // CHUNK
    nch = half_ch // NS
    stripe_ch = ROWS_PER_TILE // CHUNK

    def body(dst_hbm, out0, out1, dst_all, ones_v, buf_v, acc_sh, sem):
        c = lax.axis_index("c")
        t = lax.axis_index("s")

        pltpu.sync_copy(dst_hbm.at[pl.ds((c * NS + t) * nch, nch)], dst_all)

        one = jnp.full((L,), 1.0, jnp.float32)
        zero = jnp.zeros((L,), jnp.float32)

        def init_ones(i, _):
            for j in range(DH // L):
                ones_v[i, pl.ds(j * L, L)] = one
                buf_v[i, pl.ds(j * L, L)] = zero
            return 0
        lax.fori_loop(0, CHUNK, init_ones, 0)

        for j in range(stripe_ch):
            pltpu.sync_copy(
                buf_v, acc_sh.at[pl.ds(t * ROWS_PER_TILE + j * CHUNK, CHUNK)])
        plsc.subcore_barrier()

        def fire(k, _):
            pltpu.async_copy(ones_v, acc_sh.at[dst_all.at[k]], sem, add=True)
            return 0
        lax.fori_loop(0, nch, fire, 0)

        def drain(k, _):
            pltpu.make_async_copy(ones_v, acc_sh.at[dst_all.at[k]], sem).wait()
            return 0
        lax.fori_loop(0, nch, drain, 0)
        plsc.subcore_barrier()

        def make_out(out_hbm):
            def copy_out(j, _):
                base = t * ROWS_PER_TILE + j * CHUNK
                pltpu.sync_copy(acc_sh.at[pl.ds(base, CHUNK)], buf_v)
                pltpu.sync_copy(buf_v, out_hbm.at[pl.ds(base, CHUNK)])
                return 0
            return copy_out

        @pl.when(c == 0)
        def _():
            lax.fori_loop(0, stripe_ch, make_out(out0), 0)

        @pl.when(c == 1)
        def _():
            lax.fori_loop(0, stripe_ch, make_out(out1), 0)

    f = pl.kernel(
        body,
        out_type=(jax.ShapeDtypeStruct((NP, DH), jnp.float32),
                  jax.ShapeDtypeStruct((NP, DH), jnp.float32)),
        mesh=_mesh(),
        scratch_types=[
            pltpu.VMEM((EP // NC // NS // CHUNK, CHUNK), jnp.int32),
            pltpu.VMEM((CHUNK, DH), jnp.float32),
            pltpu.VMEM((CHUNK, DH), jnp.float32),
            pltpu.VMEM_SHARED((NP, DH), jnp.float32),
            pltpu.SemaphoreType.DMA,
        ],
    )
    return f(dst2d)


def _sc_scatter(h0, h1, src2d, dst2d):
    nch = EP // NS // CHUNK
    stripe_ch = ROWS_PER_TILE // CHUNK

    def body(h0_hbm, h1_hbm, src_hbm, dst_hbm, out0, out1,
             src_all, dst_all, bufs, acc_sh, gsems, ssems):
        c = lax.axis_index("c")
        t = lax.axis_index("s")

        zero = jnp.zeros((L,), jnp.float32)

        def init_rows(i, _):
            for j in range(DH // L):
                bufs[0][i, pl.ds(j * L, L)] = zero
            return 0
        lax.fori_loop(0, CHUNK, init_rows, 0)
        for j in range(stripe_ch):
            pltpu.async_copy(
                bufs[0],
                acc_sh.at[pl.ds(t * ROWS_PER_TILE + j * CHUNK, CHUNK)],
                gsems[0])
        for j in range(stripe_ch):
            pltpu.make_async_copy(
                bufs[0],
                acc_sh.at[pl.ds(t * ROWS_PER_TILE + j * CHUNK, CHUNK)],
                gsems[0]).wait()
        plsc.subcore_barrier()

        seg = nch // NSEG

        def make_loop(h_hbm):
            def _loop():
                for s in range(NSEG):
                    pltpu.sync_copy(
                        src_hbm.at[pl.ds(t * nch + s * seg, seg)], src_all)
                    pltpu.sync_copy(
                        dst_hbm.at[pl.ds(t * nch + s * seg, seg)], dst_all)
                    for b in range(NBUF):
                        pltpu.async_copy(
                            h_hbm.at[src_all.at[b]], bufs[b], gsems[b])

                    def step(g, _):
                        k0 = g * NBUF
                        for b in range(NBUF):
                            pltpu.make_async_copy(
                                h_hbm.at[src_all.at[k0 + b]],
                                bufs[b], gsems[b]).wait()
                            pltpu.sync_copy(
                                bufs[b], acc_sh.at[dst_all.at[k0 + b]],
                                add=True)

                            @pl.when(k0 + b + NBUF < seg)
                            def _():
                                pltpu.async_copy(
                                    h_hbm.at[src_all.at[k0 + b + NBUF]],
                                    bufs[b], gsems[b])
                        return 0
                    lax.fori_loop(0, seg // NBUF, step, 0)
            return _loop

        @pl.when(c == 0)
        def _():
            make_loop(h0_hbm)()

        @pl.when(c == 1)
        def _():
            make_loop(h1_hbm)()

        plsc.subcore_barrier()

        def make_out(out_hbm):
            def rd(j, b):
                base = t * ROWS_PER_TILE + j * CHUNK
                pltpu.async_copy(
                    acc_sh.at[pl.ds(base, CHUNK)], bufs[b], gsems[b])

            rd(0, 0)
            rd(1, 1)
            for j in range(stripe_ch):
                b = j & 1
                base = t * ROWS_PER_TILE + j * CHUNK
                pltpu.make_async_copy(
                    acc_sh.at[pl.ds(base, CHUNK)], bufs[b], gsems[b]).wait()
                pltpu.sync_copy(bufs[b], out_hbm.at[pl.ds(base, CHUNK)])
                if j + 2 < stripe_ch:
                    rd(j + 2, b)

        @pl.when(c == 0)
        def _():
            make_out(out0)

        @pl.when(c == 1)
        def _():
            make_out(out1)

    f = pl.kernel(
        body,
        out_type=(jax.ShapeDtypeStruct((NP, DH), jnp.float32),
                  jax.ShapeDtypeStruct((NP, DH), jnp.float32)),
        mesh=_mesh(),
        scratch_types=[
            pltpu.VMEM((EP // NS // CHUNK // NSEG, CHUNK), jnp.int32),
            pltpu.VMEM((EP // NS // CHUNK // NSEG, CHUNK), jnp.int32),
            [pltpu.VMEM((CHUNK, DH), jnp.float32) for _ in range(NBUF)],
            pltpu.VMEM_SHARED((NP, DH), jnp.float32),
            [pltpu.SemaphoreType.DMA for _ in range(NBUF)],
            [pltpu.SemaphoreType.DMA for _ in range(NBUF)],
        ],
    )
    return f(h0, h1, src2d, dst2d)


def _dis(dp0_blk, dp1_blk):
    deg = 1.0 + dp0_blk[:, 0:1] + dp1_blk[:, 0:1]
    return lax.rsqrt(deg)


def _tc_layer1_body(x_ref, w_ref, dp0_ref, dp1_ref, o0_ref, o1_ref):
    s = _dis(dp0_ref[...], dp1_ref[...])
    xh = jnp.dot(x_ref[...], w_ref[...], preferred_element_type=jnp.float32)
    xs = xh * s
    o0_ref[...] = xs[:, :DH]
    o1_ref[...] = xs[:, DH:]


def _tc_layer1(xp, W1, dp0, dp1):
    grid = (NP // BR,)
    return pl.pallas_call(
        _tc_layer1_body,
        grid=grid,
        in_specs=[
            pl.BlockSpec((BR, 256), lambda i: (i, 0)),
            pl.BlockSpec((256, 256), lambda i: (0, 0)),
            pl.BlockSpec((BR, DH), lambda i: (i, 0)),
            pl.BlockSpec((BR, DH), lambda i: (i, 0)),
        ],
        out_specs=(pl.BlockSpec((BR, DH), lambda i: (i, 0)),
                   pl.BlockSpec((BR, DH), lambda i: (i, 0))),
        out_shape=(jax.ShapeDtypeStruct((NP, DH), jnp.float32),
                   jax.ShapeDtypeStruct((NP, DH), jnp.float32)),
    )(xp, W1, dp0, dp1)


def _tc_mid_body(s0_ref, s1_ref, x0_ref, x1_ref, dp0_ref, dp1_ref, b_ref,
                 o0_ref, o1_ref):
    s = _dis(dp0_ref[...], dp1_ref[...])
    pre0 = s * (s0_ref[...] + x0_ref[...]) + b_ref[0:1, :]
    pre1 = s * (s1_ref[...] + x1_ref[...]) + b_ref[1:2, :]
    o0_ref[...] = s * jnp.maximum(pre0, 0.0)
    o1_ref[...] = s * jnp.maximum(pre1, 0.0)


def _tc_mid(s0, s1, x0, x1, dp0, dp1, b1):
    grid = (NP // BR,)
    b2 = b1.reshape(2, DH)
    blk = lambda i: (i, 0)
    return pl.pallas_call(
        _tc_mid_body,
        grid=grid,
        in_specs=[
            pl.BlockSpec((BR, DH), blk),
            pl.BlockSpec((BR, DH), blk),
            pl.BlockSpec((BR, DH), blk),
            pl.BlockSpec((BR, DH), blk),
            pl.BlockSpec((BR, DH), blk),
            pl.BlockSpec((BR, DH), blk),
            pl.BlockSpec((2, DH), lambda i: (0, 0)),
        ],
        out_specs=(pl.BlockSpec((BR, DH), blk),
                   pl.BlockSpec((BR, DH), blk)),
        out_shape=(jax.ShapeDtypeStruct((NP, DH), jnp.float32),
                   jax.ShapeDtypeStruct((NP, DH), jnp.float32)),
    )(s0, s1, x0, x1, dp0, dp1, b2)


def _tc_final_body(t0_ref, t1_ref, h0_ref, h1_ref, dp0_ref, dp1_ref,
                   wmu_ref, bmu_ref, wlv_ref, blv_ref, mu_ref, lv_ref):
    s = _dis(dp0_ref[...], dp1_ref[...])
    c0 = s * (t0_ref[...] + h0_ref[...])
    c1 = s * (t1_ref[...] + h1_ref[...])
    wmu = wmu_ref[...]
    wlv = wlv_ref[...]
    mu_ref[...] = (jnp.dot(c0, wmu[:DH], preferred_element_type=jnp.float32)
                   + jnp.dot(c1, wmu[DH:], preferred_element_type=jnp.float32)
                   + bmu_ref[0:1, :])
    lv_ref[...] = (jnp.dot(c0, wlv[:DH], preferred_element_type=jnp.float32)
                   + jnp.dot(c1, wlv[DH:], preferred_element_type=jnp.float32)
                   + blv_ref[0:1, :])


def _tc_final(t0, t1, h0, h1, dp0, dp1, W_mu, b_mu, W_logvar, b_logvar):
    grid = (NP // BR,)
    blk = lambda i: (i, 0)
    return pl.pallas_call(
        _tc_final_body,
        grid=grid,
        in_specs=[
            pl.BlockSpec((BR, DH), blk),
            pl.BlockSpec((BR, DH), blk),
            pl.BlockSpec((BR, DH), blk),
            pl.BlockSpec((BR, DH), blk),
            pl.BlockSpec((BR, DH), blk),
            pl.BlockSpec((BR, DH), blk),
            pl.BlockSpec((256, DH), lambda i: (0, 0)),
            pl.BlockSpec((1, DH), lambda i: (0, 0)),
            pl.BlockSpec((256, DH), lambda i: (0, 0)),
            pl.BlockSpec((1, DH), lambda i: (0, 0)),
        ],
        out_specs=(pl.BlockSpec((BR, DH), blk),
                   pl.BlockSpec((BR, DH), blk)),
        out_shape=(jax.ShapeDtypeStruct((NP, DH), jnp.float32),
                   jax.ShapeDtypeStruct((NP, DH), jnp.float32)),
    )(t0, t1, h0, h1, dp0, dp1, W_mu, b_mu.reshape(1, DH),
      W_logvar, b_logvar.reshape(1, DH))


def kernel(x, edge_index, W1, b1, W_mu, b_mu, W_logvar, b_logvar):
    src = edge_index[0].astype(jnp.int32)
    dst = edge_index[1].astype(jnp.int32)
    pad = EP - E
    src2d = jnp.concatenate([src, jnp.zeros((pad,), jnp.int32)]).reshape(
        EP // CHUNK, CHUNK)
    dst2d = jnp.concatenate([dst, jnp.full((pad,), N, jnp.int32)]).reshape(
        EP // CHUNK, CHUNK)
    xp = jnp.concatenate([x, jnp.zeros((NP - N, x.shape[1]), x.dtype)])

    dp0, dp1 = _sc_degree(dst2d)
    x0, x1 = _tc_layer1(xp, W1, dp0, dp1)
    s0, s1 = _sc_scatter(x0, x1, src2d, dst2d)
    h0, h1 = _tc_mid(s0, s1, x0, x1, dp0, dp1, b1)
    t0, t1 = _sc_scatter(h0, h1, src2d, dst2d)
    mu, lv = _tc_final(t0, t1, h0, h1, dp0, dp1, W_mu, b_mu, W_logvar, b_logvar)
    return mu[:N], lv[:N]
